# baseline (speedup 1.0000x reference)
import functools

import jax
import jax.numpy as jnp
from jax.experimental import pallas as pl
from jax.experimental.pallas import tpu as pltpu

_EB = 512
_NB = 512


def _dg(a, b):
    return jax.lax.dot_general(a, b, (((1,), (0,)), ((), ())),
                               preferred_element_type=jnp.float32)


def _dgT(a, b):
    return jax.lax.dot_general(a, b, (((0,), (0,)), ((), ())),
                               preferred_element_type=jnp.float32)


def _dgR(a, b):
    return jax.lax.dot_general(a, b, (((1,), (1,)), ((), ())),
                               preferred_element_type=jnp.float32)


def _body(nEB, nNB, L, H, Dh, D,
          emb_ref, row_ref, col_ref, w_ref, vals_ref, we_ref, wa_ref,
          out_ref, ent_ref, etab_ref, tab_ref, x_ref, p_ref, hf_ref):
    Eb, Nb = _EB, _NB
    iotaN = jax.lax.broadcasted_iota(jnp.int32, (Nb, 1), 0)

    def onehot(idx_row, nb):
        return jnp.where(iotaN + nb * Nb == idx_row, 1.0, 0.0).astype(jnp.float32)

    tab_ref[...] = jnp.zeros_like(tab_ref)
    ent_ref[...] = jnp.zeros_like(ent_ref)

    p_ref[...] = w_ref[...] * jnp.exp(vals_ref[...])

    def _zscat(eb, _):
        pv = p_ref[pl.ds(eb, 1), :]
        rr = row_ref[pl.ds(eb, 1), :]

        def inner(nb, _2):
            ohr = onehot(rr, nb)
            tab_ref[pl.ds(nb * Nb, Nb), 0:1] += _dgR(ohr, pv)
            return 0
        return jax.lax.fori_loop(0, nNB, inner, 0)
    jax.lax.fori_loop(0, nEB, _zscat, 0)

    def _entacc(eb, _):
        pv = p_ref[pl.ds(eb, 1), :]
        rr = row_ref[pl.ds(eb, 1), :]
        cc = col_ref[pl.ds(eb, 1), :]

        def g1(nb, carry):
            zg, g = carry
            ohr = onehot(rr, nb)
            ohc = onehot(cc, nb)
            zg = zg + _dgT(tab_ref[pl.ds(nb * Nb, Nb), 0:1], ohr)
            g = g + _dgT(ohc, emb_ref[pl.ds(nb * Nb, Nb), :])
            return zg, g
        zg, g = jax.lax.fori_loop(
            0, nNB, g1,
            (jnp.zeros((1, Eb), jnp.float32), jnp.zeros((Eb, D), jnp.float32)))
        cf = pv / jnp.maximum(zg, 1e-30)

        def sc(nb, _2):
            ohr = onehot(rr, nb) * cf
            ent_ref[pl.ds(nb * Nb, Nb), :] += _dg(ohr, g)
            return 0
        return jax.lax.fori_loop(0, nNB, sc, 0)
    jax.lax.fori_loop(0, nEB, _entacc, 0)

    for l in range(L):
        hf_ref[...] = jnp.zeros_like(hf_ref)
        for h in range(H):
            k = l * H + h
            for nb in range(nNB):
                ec = jnp.maximum(ent_ref[pl.ds(nb * Nb, Nb), h * Dh:(h + 1) * Dh], 0.0)
                etab_ref[pl.ds(nb * Nb, Nb), :] = ec
                wev = jnp.maximum(_dg(ec, we_ref[k]), 0.0)
                tab_ref[pl.ds(nb * Nb, Nb), 1:2] = _dg(wev, wa_ref[k, 0:Dh, :])
                tab_ref[pl.ds(nb * Nb, Nb), 2:3] = _dg(wev, wa_ref[k, Dh:2 * Dh, :])
                tab_ref[pl.ds(nb * Nb, Nb), 3:4] = jnp.zeros((Nb, 1), jnp.float32)

            def _pa(eb, _):
                rr = row_ref[pl.ds(eb, 1), :]
                cc = col_ref[pl.ds(eb, 1), :]

                def gsn(nb, carry):
                    sg, ng = carry
                    sg = sg + _dgT(tab_ref[pl.ds(nb * Nb, Nb), 1:2], onehot(rr, nb))
                    ng = ng + _dgT(tab_ref[pl.ds(nb * Nb, Nb), 2:3], onehot(cc, nb))
                    return sg, ng
                sg, ng = jax.lax.fori_loop(
                    0, nNB, gsn,
                    (jnp.zeros((1, Eb), jnp.float32), jnp.zeros((1, Eb), jnp.float32)))
                xv = sg + ng
                x_ref[pl.ds(eb, 1), :] = jnp.where(xv >= 0.0, xv, 0.3 * xv)
                return 0
            jax.lax.fori_loop(0, nEB, _pa, 0)

            xv = x_ref[...]
            M = jnp.max(xv)
            ea = jnp.exp(xv - M)
            Sg = jnp.sum(w_ref[...] * ea)
            p_ref[...] = w_ref[...] * jnp.exp(ea / Sg)

            def _z2(eb, _):
                pv = p_ref[pl.ds(eb, 1), :]
                rr = row_ref[pl.ds(eb, 1), :]

                def inner(nb, _2):
                    tab_ref[pl.ds(nb * Nb, Nb), 3:4] += _dgR(onehot(rr, nb), pv)
                    return 0
                return jax.lax.fori_loop(0, nNB, inner, 0)
            jax.lax.fori_loop(0, nEB, _z2, 0)

            def _pd(eb, _):
                pv = p_ref[pl.ds(eb, 1), :]
                rr = row_ref[pl.ds(eb, 1), :]
                cc = col_ref[pl.ds(eb, 1), :]

                def g1(nb, carry):
                    zg, g = carry
                    zg = zg + _dgT(tab_ref[pl.ds(nb * Nb, Nb), 3:4], onehot(rr, nb))
                    g = g + _dgT(onehot(cc, nb), etab_ref[pl.ds(nb * Nb, Nb), :])
                    return zg, g
                zg, g = jax.lax.fori_loop(
                    0, nNB, g1,
                    (jnp.zeros((1, Eb), jnp.float32), jnp.zeros((Eb, Dh), jnp.float32)))
                cf = pv / jnp.maximum(zg, 1e-30)

                def sc(nb, _2):
                    ohr = onehot(rr, nb) * cf
                    hf_ref[pl.ds(nb * Nb, Nb), h * Dh:(h + 1) * Dh] += _dg(ohr, g)
                    return 0
                return jax.lax.fori_loop(0, nNB, sc, 0)
            jax.lax.fori_loop(0, nEB, _pd, 0)

        for nb in range(nNB):
            t = jnp.tanh(hf_ref[pl.ds(nb * Nb, Nb), :])
            ent_ref[pl.ds(nb * Nb, Nb), :] = t
            out_ref[pl.ds(nb * Nb, Nb), l * D:(l + 1) * D] = t


def kernel(ent_emb, concept_rel, concept_attr, edge_index, edge_value,
           W_ent_attn, W_ent, W_crel, W_cattr, W_cattn):
    N, D = ent_emb.shape
    E = edge_index.shape[0]
    L, H = W_ent.shape[0], W_ent.shape[1]
    Dh = D // H

    Eb, Nb = _EB, _NB
    Np = ((N + Nb - 1) // Nb) * Nb
    Ep = ((E + Eb - 1) // Eb) * Eb
    nEB, nNB = Ep // Eb, Np // Nb

    r = edge_index[:, 0].astype(jnp.int32)
    c = edge_index[:, 1].astype(jnp.int32)
    keys = r * N + c
    head = jnp.concatenate([jnp.ones((1,), bool), keys[1:] != keys[:-1]])
    last = jnp.concatenate([keys[:-1] != keys[1:], jnp.ones((1,), bool)])
    A = jnp.cumsum(edge_value)
    B = jax.lax.cummax(jnp.where(head, A - edge_value, -1.0))
    iot = jnp.arange(E, dtype=jnp.int32)
    hidx = jax.lax.cummax(jnp.where(head, iot, -1))
    cnt = (iot - hidx + 1).astype(jnp.float32)
    vals = (A - B) / cnt
    w = last.astype(jnp.float32)

    pad = Ep - E
    r = jnp.pad(r, (0, pad)).reshape(nEB, Eb)
    c = jnp.pad(c, (0, pad)).reshape(nEB, Eb)
    w = jnp.pad(w, (0, pad)).reshape(nEB, Eb)
    vals = jnp.pad(vals, (0, pad)).reshape(nEB, Eb)
    emb = jnp.pad(ent_emb, ((0, Np - N), (0, 0)))
    we = W_ent.reshape(L * H, Dh, Dh)
    wa = W_ent_attn.reshape(L * H, 2 * Dh, 1)

    out = pl.pallas_call(
        functools.partial(_body, nEB, nNB, L, H, Dh, D),
        out_shape=jax.ShapeDtypeStruct((Np, L * D), jnp.float32),
        scratch_shapes=[
            pltpu.VMEM((Np, D), jnp.float32),
            pltpu.VMEM((Np, Dh), jnp.float32),
            pltpu.VMEM((Np, 128), jnp.float32),
            pltpu.VMEM((nEB, Eb), jnp.float32),
            pltpu.VMEM((nEB, Eb), jnp.float32),
            pltpu.VMEM((Np, D), jnp.float32),
        ],
    )(emb, r, c, w, vals, we, wa)
    return out[:N]
